# baseline (speedup 1.0000x reference)
import functools

import jax
import jax.numpy as jnp
from jax import lax
from jax.experimental import pallas as pl
from jax.experimental.pallas import tpu as pltpu
from jax.experimental.pallas import tpu_sc as plsc

_LANES = 16
_WCOLS = 3
_RING = 2


@functools.lru_cache(maxsize=None)
def _make_gather(batch: int, vocab: int, dim: int):
    info = plsc.get_sparse_core_info()
    nc, ns = info.num_cores, info.num_subcores
    nw = nc * ns
    L = _LANES
    assert batch % L == 0 and dim <= 128
    tcols = -(-vocab // 128)
    cols_per_tile = -(-tcols // nw)
    wa = _WCOLS * 128
    nwin_static = -(-cols_per_tile // _WCOLS)
    if nwin_static % 2:
        nwin_static += 1
    tail_start = (vocab // 128) * 128 if vocab % 128 else vocab
    tail_n = vocab - tail_start
    max_sk = ((vocab - wa) // 128) * 128
    n_groups_all = batch // L
    mesh = plsc.VectorSubcoreMesh(core_axis_name="c", subcore_axis_name="s")

    tail_shape = (dim, tail_n if tail_n else 128)

    @functools.partial(
        pl.kernel,
        mesh=mesh,
        out_type=jax.ShapeDtypeStruct((batch + L, 128), jnp.float32),
        scratch_types=[
            pltpu.VMEM((batch + _LANES,), jnp.int32),
            pltpu.VMEM((batch + _LANES,), jnp.int32),
            pltpu.VMEM((batch + _LANES,), jnp.int32),
            pltpu.VMEM((batch + _LANES,), jnp.int32),
            pltpu.VMEM((dim, wa), jnp.float32),
            pltpu.VMEM((dim, wa), jnp.float32),
            pltpu.VMEM(tail_shape, jnp.float32),
            pltpu.VMEM((_RING * L, 128), jnp.float32),
            pltpu.SemaphoreType.DMA,
            pltpu.SemaphoreType.DMA,
            pltpu.SemaphoreType.DMA,
        ],
        compiler_params=pltpu.CompilerParams(
            use_tc_tiling_on_sc=True, needs_layout_passes=False
        ),
    )
    def gather_kernel(
        table_hbm, idx_hbm, tail_hbm, out_hbm,
        hits_a, hits_b, wa_v, wb_v, win0, win1, tail_v, outstage,
        sem_a, sem_b, sem_out,
    ):
        wid = lax.axis_index("s") * nc + lax.axis_index("c")
        c0 = wid * cols_per_tile
        c1 = jnp.minimum(c0 + cols_per_tile, tcols)
        a0 = c0 * 128
        a1 = jnp.minimum(c1 * 128, vocab)
        my_nwin = -(-(c1 - c0) // _WCOLS)
        lane = lax.iota(jnp.int32, L)

        pltpu.sync_copy(idx_hbm, wa_v.at[pl.ds(0, batch)])
        if tail_n:
            pltpu.sync_copy(tail_hbm, tail_v)

        def scan_body(i, ptr):
            g = wa_v[pl.ds(i * L, L)]
            mi = ((g >= a0) & (g < a1)).astype(jnp.int32)
            rank = plsc.cumsum(mi) - mi
            dest = mi * (ptr + rank) + (1 - mi) * (batch + lane)
            plsc.store_scatter(hits_a, [dest], g)
            plsc.store_scatter(hits_b, [dest], lane + i * L)
            return ptr + jnp.sum(mi)

        n_hits = lax.fori_loop(0, n_groups_all, scan_body, 0)
        n_groups = -(-n_hits // L)

        def fire(k, buf, sem):
            @pl.when(k < my_nwin)
            def _():
                sk = jnp.minimum((c0 + k * _WCOLS) * 128, max_sk)
                sk = pl.multiple_of(sk, 128)
                pltpu.async_copy(table_hbm.at[:, pl.ds(sk, wa)], buf, sem)

        def drain_win(k, buf, sem):
            @pl.when(k < my_nwin)
            def _():
                pltpu.make_async_copy(table_hbm.at[:, pl.ds(0, wa)], buf, sem).wait()

        def extract(src_ref, width, base, av, bv, mi, state):
            in_flight, slot = state

            @pl.when(in_flight >= _RING)
            def _():
                pltpu.make_async_copy(
                    out_hbm.at[pl.ds(0, L)], outstage.at[pl.ds(0, L)], sem_out
                ).wait()

            off = jnp.clip(av - base, 0, width - 1)
            row0 = pl.multiple_of(slot * L, L)
            for f in range(4):
                fvec = jnp.full((L,), f, jnp.int32)
                vals = plsc.load_gather(src_ref, [fvec, off])
                plsc.store_scatter(outstage, [row0 + lane, fvec], vals)
            rows = mi * bv + (1 - mi) * (batch + lane)
            pltpu.async_copy(
                outstage.at[pl.ds(row0, L)], out_hbm.at[rows], sem_out
            )
            return (jnp.minimum(in_flight + 1, _RING), (slot + 1) % _RING)

        def compact_extract(wlo, whi, buf, width, base, state):
            def cgrp(j, cnt):
                av = hits_a[pl.ds(j * L, L)]
                bv = hits_b[pl.ds(j * L, L)]
                mi = (
                    (av >= wlo) & (av < whi) & (lane < (n_hits - j * L))
                ).astype(jnp.int32)
                n = jnp.sum(mi)

                def do(c):
                    rank = plsc.cumsum(mi) - mi
                    dest = mi * (c + rank) + (1 - mi) * (batch + lane)
                    plsc.store_scatter(wa_v, [dest], av)
                    plsc.store_scatter(wb_v, [dest], bv)
                    return c + n

                return lax.cond(n > 0, do, lambda c: c, cnt)

            cw = lax.fori_loop(0, n_groups, cgrp, 0)

            def egrp(e, st):
                av = wa_v[pl.ds(e * L, L)]
                bv = wb_v[pl.ds(e * L, L)]
                mi = (lane < (cw - e * L)).astype(jnp.int32)
                return extract(buf, width, base, av, bv, mi, st)

            return lax.cond(
                cw > 0,
                lambda st: lax.fori_loop(0, -(-cw // L), egrp, st),
                lambda st: st,
                state,
            )

        def process(k, buf, state):
            wlo = a0 + k * wa
            whi = jnp.minimum(jnp.minimum(wlo + wa, a1), tail_start)
            sk = jnp.minimum((c0 + k * _WCOLS) * 128, max_sk)
            return compact_extract(wlo, whi, buf, wa, sk, state)

        def pair(t, state):
            k0, k1 = 2 * t, 2 * t + 1
            fire(k1, win1, sem_b)
            drain_win(k0, win0, sem_a)
            state = lax.cond(
                k0 < my_nwin, lambda s: process(k0, win0, s), lambda s: s, state
            )
            fire(k1 + 1, win0, sem_a)
            drain_win(k1, win1, sem_b)
            state = lax.cond(
                k1 < my_nwin, lambda s: process(k1, win1, s), lambda s: s, state
            )
            return state

        fire(0, win0, sem_a)
        state = lax.fori_loop(0, nwin_static // 2, pair, (0, 0))

        if tail_n:
            state = compact_extract(
                tail_start, vocab, tail_v, tail_n, tail_start, state
            )

        in_flight, _ = state

        def fin(i, x):
            pltpu.make_async_copy(
                out_hbm.at[pl.ds(0, L)], outstage.at[pl.ds(0, L)], sem_out
            ).wait()
            return x

        lax.fori_loop(0, in_flight, fin, 0)

    def run(table_t, idx, tail_t):
        return gather_kernel(table_t, idx, tail_t)

    return run, tail_start, tail_n


def kernel(input, action_embedding):
    batch = input.shape[0]
    vocab, dim = action_embedding.shape
    run, tail_start, tail_n = _make_gather(batch, vocab, dim)
    idx = input.astype(jnp.int32)
    table_t = action_embedding.T
    if tail_n:
        tail_t = action_embedding[tail_start:].T
    else:
        tail_t = jnp.zeros((dim, 128), jnp.float32)
    out128 = run(table_t, idx, tail_t)
    return out128[:batch, :dim]


# speedup vs baseline: 1.0878x; 1.0878x over previous
import functools

import jax
import jax.numpy as jnp
from jax import lax
from jax.experimental import pallas as pl
from jax.experimental.pallas import tpu as pltpu
from jax.experimental.pallas import tpu_sc as plsc

_LANES = 16
_WCOLS = 4
_RING = 8


@functools.lru_cache(maxsize=None)
def _make_gather(batch: int, vocab: int, dim: int):
    info = plsc.get_sparse_core_info()
    nc, ns = info.num_cores, info.num_subcores
    nw = nc * ns
    L = _LANES
    assert batch % L == 0 and dim <= 128
    tcols = -(-vocab // 128)
    cols_per_tile = -(-tcols // nw)
    wa = _WCOLS * 128
    nwin_static = -(-cols_per_tile // _WCOLS)
    if nwin_static % 2:
        nwin_static += 1
    tail_start = (vocab // 128) * 128 if vocab % 128 else vocab
    tail_n = vocab - tail_start
    max_sk = ((vocab - wa) // 128) * 128
    n_groups_all = batch // L
    bshift = max(batch - 1, 1).bit_length()
    bmask = (1 << bshift) - 1
    assert (cols_per_tile * 128) << bshift < 2**31
    mesh = plsc.VectorSubcoreMesh(core_axis_name="c", subcore_axis_name="s")

    tail_shape = (dim, tail_n if tail_n else 128)

    @functools.partial(
        pl.kernel,
        mesh=mesh,
        out_type=jax.ShapeDtypeStruct((batch + L, 128), jnp.float32),
        scratch_types=[
            pltpu.VMEM((batch + _LANES,), jnp.int32),
            pltpu.VMEM((batch + _LANES,), jnp.int32),
            pltpu.VMEM((dim, wa), jnp.float32),
            pltpu.VMEM((dim, wa), jnp.float32),
            pltpu.VMEM(tail_shape, jnp.float32),
            pltpu.VMEM((_RING * L, 128), jnp.float32),
            pltpu.SemaphoreType.DMA,
            pltpu.SemaphoreType.DMA,
            pltpu.SemaphoreType.DMA,
        ],
        compiler_params=pltpu.CompilerParams(
            use_tc_tiling_on_sc=True, needs_layout_passes=False
        ),
    )
    def gather_kernel(
        table_hbm, idx_hbm, tail_hbm, out_hbm,
        hits_p, wp_v, win0, win1, tail_v, outstage,
        sem_a, sem_b, sem_out,
    ):
        wid = lax.axis_index("s") * nc + lax.axis_index("c")
        c0 = wid * cols_per_tile
        c1 = jnp.minimum(c0 + cols_per_tile, tcols)
        a0 = c0 * 128
        a1 = jnp.minimum(c1 * 128, vocab)
        my_nwin = -(-(c1 - c0) // _WCOLS)
        lane = lax.iota(jnp.int32, L)

        pltpu.sync_copy(idx_hbm, wp_v.at[pl.ds(0, batch)])
        if tail_n:
            pltpu.sync_copy(tail_hbm, tail_v)

        def scan_body(i, ptr):
            g = wp_v[pl.ds(i * L, L)]
            mi = ((g >= a0) & (g < a1)).astype(jnp.int32)
            n = jnp.sum(mi)

            def do(p):
                rank = plsc.cumsum(mi) - mi
                dest = mi * (p + rank) + (1 - mi) * (batch + lane)
                packed = ((g - a0) << bshift) + (lane + i * L)
                plsc.store_scatter(hits_p, [dest], packed)
                return p + n

            return lax.cond(n > 0, do, lambda p: p, ptr)

        n_hits = lax.fori_loop(0, n_groups_all, scan_body, 0)
        n_groups = -(-n_hits // L)

        def fire(k, buf, sem):
            @pl.when(k < my_nwin)
            def _():
                sk = jnp.minimum((c0 + k * _WCOLS) * 128, max_sk)
                sk = pl.multiple_of(sk, 128)
                pltpu.async_copy(table_hbm.at[:, pl.ds(sk, wa)], buf, sem)

        def drain_win(k, buf, sem):
            @pl.when(k < my_nwin)
            def _():
                pltpu.make_async_copy(table_hbm.at[:, pl.ds(0, wa)], buf, sem).wait()

        def extract(src_ref, width, base_rel, hp, mi, state):
            in_flight, slot = state

            @pl.when(in_flight >= _RING)
            def _():
                pltpu.make_async_copy(
                    out_hbm.at[pl.ds(0, L)], outstage.at[pl.ds(0, L)], sem_out
                ).wait()

            av = lax.shift_right_logical(hp, bshift)
            bv = hp & bmask
            off = jnp.clip(av - base_rel, 0, width - 1)
            row0 = pl.multiple_of(slot * L, L)
            for f in range(dim):
                fvec = jnp.full((L,), f, jnp.int32)
                vals = plsc.load_gather(src_ref, [fvec, off])
                plsc.store_scatter(outstage, [row0 + lane, fvec], vals)
            rows = mi * bv + (1 - mi) * (batch + lane)
            pltpu.async_copy(
                outstage.at[pl.ds(row0, L)], out_hbm.at[rows], sem_out
            )
            return (jnp.minimum(in_flight + 1, _RING), (slot + 1) % _RING)

        def compact_extract(wlo_rel, whi_rel, buf, width, base_rel, state):
            plo = wlo_rel << bshift
            phi = whi_rel << bshift

            def cgrp(j, cnt):
                hp = hits_p[pl.ds(j * L, L)]
                mi = (
                    (hp >= plo) & (hp < phi) & (lane < (n_hits - j * L))
                ).astype(jnp.int32)
                n = jnp.sum(mi)

                def do(c):
                    rank = plsc.cumsum(mi) - mi
                    dest = mi * (c + rank) + (1 - mi) * (batch + lane)
                    plsc.store_scatter(wp_v, [dest], hp)
                    return c + n

                return lax.cond(n > 0, do, lambda c: c, cnt)

            cw = lax.fori_loop(0, n_groups, cgrp, 0)

            def egrp(e, st):
                hp = wp_v[pl.ds(e * L, L)]
                mi = (lane < (cw - e * L)).astype(jnp.int32)
                return extract(buf, width, base_rel, hp, mi, st)

            return lax.cond(
                cw > 0,
                lambda st: lax.fori_loop(0, -(-cw // L), egrp, st),
                lambda st: st,
                state,
            )

        def process(k, buf, state):
            wlo_rel = k * wa
            whi_rel = jnp.minimum(
                jnp.minimum(wlo_rel + wa, a1 - a0), tail_start - a0
            )
            sk = jnp.minimum((c0 + k * _WCOLS) * 128, max_sk)
            return compact_extract(wlo_rel, whi_rel, buf, wa, sk - a0, state)

        def pair(t, state):
            k0, k1 = 2 * t, 2 * t + 1
            fire(k1, win1, sem_b)
            drain_win(k0, win0, sem_a)
            state = lax.cond(
                k0 < my_nwin, lambda s: process(k0, win0, s), lambda s: s, state
            )
            fire(k1 + 1, win0, sem_a)
            drain_win(k1, win1, sem_b)
            state = lax.cond(
                k1 < my_nwin, lambda s: process(k1, win1, s), lambda s: s, state
            )
            return state

        fire(0, win0, sem_a)
        state = lax.fori_loop(0, nwin_static // 2, pair, (0, 0))

        if tail_n:
            state = compact_extract(
                tail_start - a0, a1 - a0, tail_v, tail_n, tail_start - a0, state
            )

        in_flight, _ = state

        def fin(i, x):
            pltpu.make_async_copy(
                out_hbm.at[pl.ds(0, L)], outstage.at[pl.ds(0, L)], sem_out
            ).wait()
            return x

        lax.fori_loop(0, in_flight, fin, 0)

    def run(table_t, idx, tail_t):
        return gather_kernel(table_t, idx, tail_t)

    return run, tail_start, tail_n


def kernel(input, action_embedding):
    batch = input.shape[0]
    vocab, dim = action_embedding.shape
    run, tail_start, tail_n = _make_gather(batch, vocab, dim)
    idx = input.astype(jnp.int32)
    table_t = action_embedding.T
    if tail_n:
        tail_t = action_embedding[tail_start:].T
    else:
        tail_t = jnp.zeros((dim, 128), jnp.float32)
    out128 = run(table_t, idx, tail_t)
    return out128[:batch, :dim]


# speedup vs baseline: 1.6606x; 1.5266x over previous
import functools

import jax
import jax.numpy as jnp
from jax import lax
from jax.experimental import pallas as pl
from jax.experimental.pallas import tpu as pltpu
from jax.experimental.pallas import tpu_sc as plsc

_LANES = 16
_WCOLS = 3
_NSLOT = 8


@functools.lru_cache(maxsize=None)
def _make_gather(batch: int, vocab: int, dim: int):
    info = plsc.get_sparse_core_info()
    nc, ns = info.num_cores, info.num_subcores
    nw = nc * ns
    L = _LANES
    assert batch % L == 0 and dim <= 128
    tcols = -(-vocab // 128)
    cols_per_tile = -(-tcols // nw)
    wa = _WCOLS * 128
    nwin_static = -(-cols_per_tile // _WCOLS)
    if nwin_static % 2:
        nwin_static += 1
    tail_start = (vocab // 128) * 128 if vocab % 128 else vocab
    tail_n = vocab - tail_start
    max_sk = ((vocab - wa) // 128) * 128
    n_groups_all = batch // L
    bshift = max(batch - 1, 1).bit_length()
    bmask = (1 << bshift) - 1
    assert (cols_per_tile * 128) << bshift < 2**31
    mesh = plsc.VectorSubcoreMesh(core_axis_name="c", subcore_axis_name="s")

    tail_shape = (dim, tail_n if tail_n else 128)

    @functools.partial(
        pl.kernel,
        mesh=mesh,
        out_type=jax.ShapeDtypeStruct((batch + 128, 128), jnp.float32),
        scratch_types=[
            pltpu.VMEM((batch + _LANES,), jnp.int32),
            pltpu.VMEM((batch + _LANES,), jnp.int32),
            pltpu.VMEM((dim, wa), jnp.float32),
            pltpu.VMEM((dim, wa), jnp.float32),
            pltpu.VMEM(tail_shape, jnp.float32),
            pltpu.VMEM((2 * _NSLOT * L, 128), jnp.float32),
            pltpu.VMEM((_NSLOT * L,), jnp.int32),
            pltpu.VMEM((_NSLOT * L,), jnp.int32),
            pltpu.SemaphoreType.DMA,
            pltpu.SemaphoreType.DMA,
            pltpu.SemaphoreType.DMA,
            pltpu.SemaphoreType.DMA,
        ],
        compiler_params=pltpu.CompilerParams(
            use_tc_tiling_on_sc=True, needs_layout_passes=False
        ),
    )
    def gather_kernel(
        table_hbm, idx_hbm, tail_hbm, out_hbm,
        hits_p, wp_v, win0, win1, tail_v, outstage, idx_a, idx_b,
        sem_a, sem_b, sem_oa, sem_ob,
    ):
        wid = lax.axis_index("s") * nc + lax.axis_index("c")
        c0 = wid * cols_per_tile
        c1 = jnp.minimum(c0 + cols_per_tile, tcols)
        a0 = c0 * 128
        a1 = jnp.minimum(c1 * 128, vocab)
        my_nwin = -(-(c1 - c0) // _WCOLS)
        lane = lax.iota(jnp.int32, L)

        pltpu.sync_copy(idx_hbm, wp_v.at[pl.ds(0, batch)])
        if tail_n:
            pltpu.sync_copy(tail_hbm, tail_v)

        def scan_body(i, ptr):
            g = wp_v[pl.ds(i * L, L)]
            mi = ((g >= a0) & (g < a1)).astype(jnp.int32)
            n = jnp.sum(mi)

            def do(p):
                rank = plsc.cumsum(mi) - mi
                dest = mi * (p + rank) + (1 - mi) * (batch + lane)
                packed = ((g - a0) << bshift) + (lane + i * L)
                plsc.store_scatter(hits_p, [dest], packed)
                return p + n

            return lax.cond(n > 0, do, lambda p: p, ptr)

        n_hits = lax.fori_loop(0, n_groups_all, scan_body, 0)
        n_groups = -(-n_hits // L)

        def fire(k, buf, sem):
            @pl.when(k < my_nwin)
            def _():
                sk = jnp.minimum((c0 + k * _WCOLS) * 128, max_sk)
                sk = pl.multiple_of(sk, 128)
                pltpu.async_copy(table_hbm.at[:, pl.ds(sk, wa)], buf, sem)

        def drain_win(k, buf, sem):
            @pl.when(k < my_nwin)
            def _():
                pltpu.make_async_copy(table_hbm.at[:, pl.ds(0, wa)], buf, sem).wait()

        def drain_half(sem):
            pltpu.make_async_copy(
                out_hbm.at[pl.ds(0, _NSLOT * L)],
                outstage.at[pl.ds(0, _NSLOT * L)],
                sem,
            ).wait()

        def fire_half(half):
            if half == 0:
                pltpu.async_copy(
                    outstage.at[pl.ds(0, _NSLOT * L)], out_hbm.at[idx_a], sem_oa
                )
            else:
                pltpu.async_copy(
                    outstage.at[pl.ds(_NSLOT * L, _NSLOT * L)],
                    out_hbm.at[idx_b],
                    sem_ob,
                )

        def extract(src_ref, width, base_rel, hp, mi, c):
            slot = lax.rem(c, _NSLOT)
            half = lax.rem(lax.div(c, _NSLOT), 2)

            @pl.when((slot == 0) & (c >= 2 * _NSLOT) & (half == 0))
            def _():
                drain_half(sem_oa)

            @pl.when((slot == 0) & (c >= 2 * _NSLOT) & (half == 1))
            def _():
                drain_half(sem_ob)

            av = lax.shift_right_logical(hp, bshift)
            bv = hp & bmask
            off = jnp.clip(av - base_rel, 0, width - 1)
            row0 = pl.multiple_of((half * _NSLOT + slot) * L, L)
            for f in range(dim):
                fvec = jnp.full((L,), f, jnp.int32)
                vals = plsc.load_gather(src_ref, [fvec, off])
                plsc.store_scatter(outstage, [row0 + lane, fvec], vals)
            rows = mi * bv + (1 - mi) * (batch + slot * L + lane)

            @pl.when(half == 0)
            def _():
                idx_a[pl.ds(pl.multiple_of(slot * L, L), L)] = rows

            @pl.when(half == 1)
            def _():
                idx_b[pl.ds(pl.multiple_of(slot * L, L), L)] = rows

            @pl.when((slot == _NSLOT - 1) & (half == 0))
            def _():
                fire_half(0)

            @pl.when((slot == _NSLOT - 1) & (half == 1))
            def _():
                fire_half(1)

            return c + 1

        def compact_extract(wlo_rel, whi_rel, buf, width, base_rel, state):
            plo = wlo_rel << bshift
            phi = whi_rel << bshift

            def cgrp(j, cnt):
                hp = hits_p[pl.ds(j * L, L)]
                mi = (
                    (hp >= plo) & (hp < phi) & (lane < (n_hits - j * L))
                ).astype(jnp.int32)
                n = jnp.sum(mi)

                def do(c):
                    rank = plsc.cumsum(mi) - mi
                    dest = mi * (c + rank) + (1 - mi) * (batch + lane)
                    plsc.store_scatter(wp_v, [dest], hp)
                    return c + n

                return lax.cond(n > 0, do, lambda c: c, cnt)

            cw = lax.fori_loop(0, n_groups, cgrp, 0)

            def egrp(e, st):
                hp = wp_v[pl.ds(e * L, L)]
                mi = (lane < (cw - e * L)).astype(jnp.int32)
                return extract(buf, width, base_rel, hp, mi, st)


            return lax.cond(
                cw > 0,
                lambda st: lax.fori_loop(0, -(-cw // L), egrp, st),
                lambda st: st,
                state,
            )

        def process(k, buf, state):
            wlo_rel = k * wa
            whi_rel = jnp.clip(
                jnp.minimum(wlo_rel + wa, tail_start - a0), 0, a1 - a0
            )
            sk = jnp.minimum((c0 + k * _WCOLS) * 128, max_sk)
            return compact_extract(wlo_rel, whi_rel, buf, wa, sk - a0, state)

        def pair(t, state):
            k0, k1 = 2 * t, 2 * t + 1
            fire(k1, win1, sem_b)
            drain_win(k0, win0, sem_a)
            state = lax.cond(
                k0 < my_nwin, lambda s: process(k0, win0, s), lambda s: s, state
            )
            fire(k1 + 1, win0, sem_a)
            drain_win(k1, win1, sem_b)
            state = lax.cond(
                k1 < my_nwin, lambda s: process(k1, win1, s), lambda s: s, state
            )
            return state

        fire(0, win0, sem_a)
        state = lax.fori_loop(0, nwin_static // 2, pair, 0)

        if tail_n:
            ts_rel = jnp.clip(tail_start - a0, 0, a1 - a0)
            state = compact_extract(
                ts_rel, a1 - a0, tail_v, tail_n, ts_rel, state
            )

        c = state
        partial = lax.rem(c, _NSLOT)
        half = lax.rem(lax.div(c, _NSLOT), 2)

        def fill(s, x):
            dummy = batch + s * L + lane

            @pl.when(half == 0)
            def _():
                idx_a[pl.ds(pl.multiple_of(s * L, L), L)] = dummy

            @pl.when(half == 1)
            def _():
                idx_b[pl.ds(pl.multiple_of(s * L, L), L)] = dummy

            return x

        @pl.when(partial > 0)
        def _():
            lax.fori_loop(partial, _NSLOT, fill, 0)

        @pl.when((partial > 0) & (half == 0))
        def _():
            fire_half(0)

        @pl.when((partial > 0) & (half == 1))
        def _():
            fire_half(1)

        fires = lax.div(c, _NSLOT) + (partial > 0).astype(jnp.int32)
        fires_a = lax.div(fires + 1, 2)
        fires_b = lax.div(fires, 2)

        @pl.when(fires_a > 0)
        def _():
            drain_half(sem_oa)

        @pl.when(fires_b > 0)
        def _():
            drain_half(sem_ob)

    def run(table_t, idx, tail_t):
        return gather_kernel(table_t, idx, tail_t)

    return run, tail_start, tail_n


def kernel(input, action_embedding):
    batch = input.shape[0]
    vocab, dim = action_embedding.shape
    run, tail_start, tail_n = _make_gather(batch, vocab, dim)
    idx = input.astype(jnp.int32)
    table_t = action_embedding.T
    if tail_n:
        tail_t = action_embedding[tail_start:].T
    else:
        tail_t = jnp.zeros((dim, 128), jnp.float32)
    out128 = run(table_t, idx, tail_t)
    return out128[:batch, :dim]


# speedup vs baseline: 1.6617x; 1.0006x over previous
import functools

import jax
import jax.numpy as jnp
from jax import lax
from jax.experimental import pallas as pl
from jax.experimental.pallas import tpu as pltpu
from jax.experimental.pallas import tpu_sc as plsc

_LANES = 16
_WCOLS = 3
_NSLOT = 8


@functools.lru_cache(maxsize=None)
def _make_gather(batch: int, vocab: int, dim: int):
    info = plsc.get_sparse_core_info()
    nc, ns = info.num_cores, info.num_subcores
    nw = nc * ns
    L = _LANES
    assert batch % L == 0 and dim <= 128
    tcols = -(-vocab // 128)
    cols_per_tile = -(-tcols // nw)
    wa = _WCOLS * 128
    nwin_static = -(-cols_per_tile // _WCOLS)
    if nwin_static % 2:
        nwin_static += 1
    tail_start = (vocab // 128) * 128 if vocab % 128 else vocab
    tail_n = vocab - tail_start
    max_sk = ((vocab - wa) // 128) * 128
    n_groups_all = batch // L
    bshift = max(batch - 1, 1).bit_length()
    bmask = (1 << bshift) - 1
    assert (cols_per_tile * 128) << bshift < 2**31
    mesh = plsc.VectorSubcoreMesh(core_axis_name="c", subcore_axis_name="s")

    tail_shape = (dim, tail_n if tail_n else 128)

    @functools.partial(
        pl.kernel,
        mesh=mesh,
        out_type=jax.ShapeDtypeStruct((batch + 128, 128), jnp.float32),
        scratch_types=[
            pltpu.VMEM((batch + _LANES,), jnp.int32),
            pltpu.VMEM((batch + _LANES,), jnp.int32),
            pltpu.VMEM((dim, wa), jnp.float32),
            pltpu.VMEM((dim, wa), jnp.float32),
            pltpu.VMEM(tail_shape, jnp.float32),
            pltpu.VMEM((2 * _NSLOT * L, 128), jnp.float32),
            pltpu.VMEM((_NSLOT * L,), jnp.int32),
            pltpu.VMEM((_NSLOT * L,), jnp.int32),
            pltpu.SemaphoreType.DMA,
            pltpu.SemaphoreType.DMA,
            pltpu.SemaphoreType.DMA,
            pltpu.SemaphoreType.DMA,
        ],
        compiler_params=pltpu.CompilerParams(
            use_tc_tiling_on_sc=True, needs_layout_passes=False
        ),
    )
    def gather_kernel(
        table_hbm, idx_hbm, tail_hbm, out_hbm,
        hits_p, wp_v, win0, win1, tail_v, outstage, idx_a, idx_b,
        sem_a, sem_b, sem_oa, sem_ob,
    ):
        wid = lax.axis_index("s") * nc + lax.axis_index("c")
        c0 = wid * cols_per_tile
        c1 = jnp.minimum(c0 + cols_per_tile, tcols)
        a0 = c0 * 128
        a1 = jnp.minimum(c1 * 128, vocab)
        my_nwin = -(-(c1 - c0) // _WCOLS)
        lane = lax.iota(jnp.int32, L)

        def fire(k, buf, sem):
            @pl.when(k < my_nwin)
            def _():
                sk = jnp.minimum((c0 + k * _WCOLS) * 128, max_sk)
                sk = pl.multiple_of(sk, 128)
                pltpu.async_copy(table_hbm.at[:, pl.ds(sk, wa)], buf, sem)

        fire(0, win0, sem_a)
        fire(1, win1, sem_b)

        pltpu.sync_copy(idx_hbm, wp_v.at[pl.ds(0, batch)])
        if tail_n:
            pltpu.sync_copy(tail_hbm, tail_v)

        def scan_body(i, ptr):
            g = wp_v[pl.ds(i * L, L)]
            mi = ((g >= a0) & (g < a1)).astype(jnp.int32)
            n = jnp.sum(mi)

            def do(p):
                rank = plsc.cumsum(mi) - mi
                dest = mi * (p + rank) + (1 - mi) * (batch + lane)
                packed = ((g - a0) << bshift) + (lane + i * L)
                plsc.store_scatter(hits_p, [dest], packed)
                return p + n

            return lax.cond(n > 0, do, lambda p: p, ptr)

        n_hits = lax.fori_loop(0, n_groups_all, scan_body, 0)
        n_groups = -(-n_hits // L)

        def drain_win(k, buf, sem):
            @pl.when(k < my_nwin)
            def _():
                pltpu.make_async_copy(table_hbm.at[:, pl.ds(0, wa)], buf, sem).wait()

        def drain_half(sem):
            pltpu.make_async_copy(
                out_hbm.at[pl.ds(0, _NSLOT * L)],
                outstage.at[pl.ds(0, _NSLOT * L)],
                sem,
            ).wait()

        def fire_half(half):
            if half == 0:
                pltpu.async_copy(
                    outstage.at[pl.ds(0, _NSLOT * L)], out_hbm.at[idx_a], sem_oa
                )
            else:
                pltpu.async_copy(
                    outstage.at[pl.ds(_NSLOT * L, _NSLOT * L)],
                    out_hbm.at[idx_b],
                    sem_ob,
                )

        def extract(src_ref, width, base_rel, hp, mi, c):
            slot = lax.rem(c, _NSLOT)
            half = lax.rem(lax.div(c, _NSLOT), 2)

            @pl.when((slot == 0) & (c >= 2 * _NSLOT) & (half == 0))
            def _():
                drain_half(sem_oa)

            @pl.when((slot == 0) & (c >= 2 * _NSLOT) & (half == 1))
            def _():
                drain_half(sem_ob)

            av = lax.shift_right_logical(hp, bshift)
            bv = hp & bmask
            off = jnp.clip(av - base_rel, 0, width - 1)
            row0 = pl.multiple_of((half * _NSLOT + slot) * L, L)
            for f in range(dim):
                fvec = jnp.full((L,), f, jnp.int32)
                vals = plsc.load_gather(src_ref, [fvec, off])
                plsc.store_scatter(outstage, [row0 + lane, fvec], vals)
            rows = mi * bv + (1 - mi) * (batch + slot * L + lane)

            @pl.when(half == 0)
            def _():
                idx_a[pl.ds(pl.multiple_of(slot * L, L), L)] = rows

            @pl.when(half == 1)
            def _():
                idx_b[pl.ds(pl.multiple_of(slot * L, L), L)] = rows

            @pl.when((slot == _NSLOT - 1) & (half == 0))
            def _():
                fire_half(0)

            @pl.when((slot == _NSLOT - 1) & (half == 1))
            def _():
                fire_half(1)

            return c + 1

        def compact_extract(wlo_rel, whi_rel, buf, width, base_rel, state):
            plo = wlo_rel << bshift
            phi = whi_rel << bshift

            def cgrp(j, cnt):
                hp = hits_p[pl.ds(j * L, L)]
                mi = (
                    (hp >= plo) & (hp < phi) & (lane < (n_hits - j * L))
                ).astype(jnp.int32)
                n = jnp.sum(mi)

                def do(c):
                    rank = plsc.cumsum(mi) - mi
                    dest = mi * (c + rank) + (1 - mi) * (batch + lane)
                    plsc.store_scatter(wp_v, [dest], hp)
                    return c + n

                return lax.cond(n > 0, do, lambda c: c, cnt)

            cw = lax.fori_loop(0, n_groups, cgrp, 0)

            def egrp(e, st):
                hp = wp_v[pl.ds(e * L, L)]
                mi = (lane < (cw - e * L)).astype(jnp.int32)
                return extract(buf, width, base_rel, hp, mi, st)


            return lax.cond(
                cw > 0,
                lambda st: lax.fori_loop(0, -(-cw // L), egrp, st),
                lambda st: st,
                state,
            )

        def process(k, buf, state):
            wlo_rel = k * wa
            whi_rel = jnp.clip(
                jnp.minimum(wlo_rel + wa, tail_start - a0), 0, a1 - a0
            )
            sk = jnp.minimum((c0 + k * _WCOLS) * 128, max_sk)
            return compact_extract(wlo_rel, whi_rel, buf, wa, sk - a0, state)

        def pair(t, state):
            k0, k1 = 2 * t, 2 * t + 1
            drain_win(k0, win0, sem_a)
            state = lax.cond(
                k0 < my_nwin, lambda s: process(k0, win0, s), lambda s: s, state
            )
            fire(k0 + 2, win0, sem_a)
            drain_win(k1, win1, sem_b)
            state = lax.cond(
                k1 < my_nwin, lambda s: process(k1, win1, s), lambda s: s, state
            )
            fire(k1 + 2, win1, sem_b)
            return state

        state = lax.fori_loop(0, nwin_static // 2, pair, 0)

        if tail_n:
            ts_rel = jnp.clip(tail_start - a0, 0, a1 - a0)
            state = compact_extract(
                ts_rel, a1 - a0, tail_v, tail_n, ts_rel, state
            )

        c = state
        partial = lax.rem(c, _NSLOT)
        half = lax.rem(lax.div(c, _NSLOT), 2)

        def fill(s, x):
            dummy = batch + s * L + lane

            @pl.when(half == 0)
            def _():
                idx_a[pl.ds(pl.multiple_of(s * L, L), L)] = dummy

            @pl.when(half == 1)
            def _():
                idx_b[pl.ds(pl.multiple_of(s * L, L), L)] = dummy

            return x

        @pl.when(partial > 0)
        def _():
            lax.fori_loop(partial, _NSLOT, fill, 0)

        @pl.when((partial > 0) & (half == 0))
        def _():
            fire_half(0)

        @pl.when((partial > 0) & (half == 1))
        def _():
            fire_half(1)

        fires = lax.div(c, _NSLOT) + (partial > 0).astype(jnp.int32)
        fires_a = lax.div(fires + 1, 2)
        fires_b = lax.div(fires, 2)

        @pl.when(fires_a > 0)
        def _():
            drain_half(sem_oa)

        @pl.when(fires_b > 0)
        def _():
            drain_half(sem_ob)

    def run(table_t, idx, tail_t):
        return gather_kernel(table_t, idx, tail_t)

    return run, tail_start, tail_n


def kernel(input, action_embedding):
    batch = input.shape[0]
    vocab, dim = action_embedding.shape
    run, tail_start, tail_n = _make_gather(batch, vocab, dim)
    idx = input.astype(jnp.int32)
    table_t = action_embedding.T
    if tail_n:
        tail_t = action_embedding[tail_start:].T
    else:
        tail_t = jnp.zeros((dim, 128), jnp.float32)
    out128 = run(table_t, idx, tail_t)
    return out128[:batch, :dim]
